# baseline (speedup 1.0000x reference)
import functools

import jax
import jax.numpy as jnp
from jax import lax
from jax.experimental import pallas as pl
from jax.experimental.pallas import tpu as pltpu
from jax.experimental.pallas import tpu_sc as plsc

B = 16384
L = 16
NC, NS = 2, 16
NW = NC * NS
BPW = B // NW
TAB = 1024

_mesh = plsc.VectorSubcoreMesh(core_axis_name="c", subcore_axis_name="s")


@functools.partial(
    pl.kernel,
    mesh=_mesh,
    out_type=jax.ShapeDtypeStruct((B,), jnp.float32),
    scratch_types=[
        pltpu.VMEM((BPW,), jnp.int32),
        pltpu.VMEM((BPW,), jnp.float32),
        pltpu.VMEM_SHARED((1001,), jnp.float32),
        pltpu.SemaphoreType.DMA,
        pltpu.SemaphoreType.DMA,
    ],
)
def _gather_kernel(t_hbm, g_hbm, out_hbm, idx_v, out_v, tab_sh, sem_i, sem_g):
    sid = lax.axis_index("s")
    wid = sid * NC + lax.axis_index("c")
    base = wid * BPW
    idx_cp = pltpu.async_copy(t_hbm.at[pl.ds(base, BPW)], idx_v, sem_i)

    @pl.when(sid == 0)
    def _stage_table():
        pltpu.sync_copy(g_hbm, tab_sh)

    plsc.subcore_barrier()
    idx_cp.wait()
    pltpu.async_copy(tab_sh.at[idx_v], out_v, sem_g).wait()
    pltpu.sync_copy(out_v, out_hbm.at[pl.ds(base, BPW)])


def kernel(t, gammas):
    return _gather_kernel(t, gammas)


# speedup vs baseline: 1.0712x; 1.0712x over previous
import functools

import jax
import jax.numpy as jnp
from jax import lax
from jax.experimental import pallas as pl
from jax.experimental.pallas import tpu as pltpu
from jax.experimental.pallas import tpu_sc as plsc

B = 16384
L = 16
NC, NS = 1, 16
NW = NC * NS
BPW = B // NW
TAB = 1024

_mesh = plsc.VectorSubcoreMesh(
    core_axis_name="c", subcore_axis_name="s", num_cores=NC
)


@functools.partial(
    pl.kernel,
    mesh=_mesh,
    out_type=jax.ShapeDtypeStruct((B,), jnp.float32),
    scratch_types=[
        pltpu.VMEM((BPW,), jnp.int32),
        pltpu.VMEM((BPW,), jnp.float32),
        pltpu.VMEM_SHARED((1001,), jnp.float32),
        pltpu.SemaphoreType.DMA,
        pltpu.SemaphoreType.DMA,
    ],
)
def _gather_kernel(t_hbm, g_hbm, out_hbm, idx_v, out_v, tab_sh, sem_i, sem_g):
    sid = lax.axis_index("s")
    wid = sid * NC + lax.axis_index("c")
    base = wid * BPW
    idx_cp = pltpu.async_copy(t_hbm.at[pl.ds(base, BPW)], idx_v, sem_i)

    @pl.when(sid == 0)
    def _stage_table():
        pltpu.sync_copy(g_hbm, tab_sh)

    plsc.subcore_barrier()
    idx_cp.wait()
    pltpu.async_copy(tab_sh.at[idx_v], out_v, sem_g).wait()
    pltpu.sync_copy(out_v, out_hbm.at[pl.ds(base, BPW)])


def kernel(t, gammas):
    return _gather_kernel(t, gammas)
